# speedup vs baseline: 2.7335x; 2.7335x over previous
import functools

import jax
import jax.numpy as jnp
import numpy as np
from jax import lax
from jax.experimental import pallas as pl
from jax.experimental.pallas import tpu as pltpu
from jax.experimental.pallas import tpu_sc as plsc

_L = 16
_NC = 2
_NS = 16
_NW = _NC * _NS


def _sinusoidal_pe_np(seq_len: int, d_model: int) -> np.ndarray:
    pos = np.arange(seq_len, dtype=np.float32)[:, None]
    i = np.arange(0, d_model, 2, dtype=np.float32)
    div = np.exp(-(np.log(10000.0)) * i / d_model)
    pe = np.zeros((seq_len, d_model), dtype=np.float32)
    pe[:, 0::2] = np.sin(pos * div)
    pe[:, 1::2] = np.cos(pos * div)
    return pe


@functools.partial(jax.jit, static_argnames=("batch", "seq", "d_model"))
def _embed(x_flat, table, pe, *, batch, seq, d_model):
    P = seq // _NW
    C = 32
    n_h = P // C
    nchunks = batch * n_h

    mesh = plsc.VectorSubcoreMesh(core_axis_name="c", subcore_axis_name="s")

    @functools.partial(
        pl.kernel,
        out_type=jax.ShapeDtypeStruct((batch * seq, d_model), jnp.float32),
        mesh=mesh,
        scratch_types=[
            pltpu.VMEM((P, d_model), jnp.float32),
            pltpu.VMEM((C, d_model), jnp.float32),
            pltpu.VMEM((C, d_model), jnp.float32),
            pltpu.VMEM((C,), jnp.int32),
            pltpu.VMEM((C,), jnp.int32),
            pltpu.SemaphoreType.DMA,
            pltpu.SemaphoreType.DMA,
        ],
    )
    def body(x_hbm, table_hbm, pe_hbm, out_hbm,
             pe_v, tok0, tok1, idx0, idx1, gsem0, gsem1):
        wid = lax.axis_index("s") * _NC + lax.axis_index("c")
        pos0 = wid * P
        pltpu.sync_copy(pe_hbm.at[pl.ds(pos0, P)], pe_v)

        toks = (tok0, tok1)
        idxs = (idx0, idx1)
        gsems = (gsem0, gsem1)

        def chunk_base(c):
            b, h = divmod(c, n_h)
            return b * seq + pos0 + h * C

        pltpu.sync_copy(x_hbm.at[pl.ds(chunk_base(0), C)], idxs[0])
        gathers = [pltpu.async_copy(table_hbm.at[idxs[0]], toks[0], gsems[0])]

        for c in range(nchunks):
            p = c % 2
            if c + 1 < nchunks:
                q = (c + 1) % 2
                pltpu.sync_copy(x_hbm.at[pl.ds(chunk_base(c + 1), C)], idxs[q])
                gathers.append(
                    pltpu.async_copy(table_hbm.at[idxs[q]], toks[q], gsems[q]))
            gathers[c].wait()
            h = c % n_h

            @pl.loop(0, C)
            def _(r):
                @pl.loop(0, d_model, step=_L)
                def _(col):
                    plsc.addupdate(toks[p].at[r, pl.ds(col, _L)],
                                   pe_v[h * C + r, pl.ds(col, _L)])

            pltpu.sync_copy(toks[p], out_hbm.at[pl.ds(chunk_base(c), C)])

    return body(x_flat, table, pe)


def kernel(x, token_table):
    batch, seq = x.shape
    d_model = token_table.shape[1]
    pe = jnp.asarray(_sinusoidal_pe_np(seq, d_model))
    x_flat = x.reshape(batch * seq).astype(jnp.int32)
    out = _embed(x_flat, token_table, pe,
                 batch=batch, seq=seq, d_model=d_model)
    return out.reshape(batch, seq, d_model)


# speedup vs baseline: 3.8241x; 1.3990x over previous
import functools

import jax
import jax.numpy as jnp
import numpy as np
from jax import lax
from jax.experimental import pallas as pl
from jax.experimental.pallas import tpu as pltpu
from jax.experimental.pallas import tpu_sc as plsc

_L = 16
_NC = 2
_NS = 16
_NW = _NC * _NS


def _sinusoidal_pe_np(seq_len: int, d_model: int) -> np.ndarray:
    pos = np.arange(seq_len, dtype=np.float32)[:, None]
    i = np.arange(0, d_model, 2, dtype=np.float32)
    div = np.exp(-(np.log(10000.0)) * i / d_model)
    pe = np.zeros((seq_len, d_model), dtype=np.float32)
    pe[:, 0::2] = np.sin(pos * div)
    pe[:, 1::2] = np.cos(pos * div)
    return pe


@functools.partial(jax.jit, static_argnames=("batch", "seq", "d_model"))
def _embed(x_flat, table, pe, *, batch, seq, d_model):
    P = seq // _NW
    C = 32
    n_h = P // C
    nchunks = batch * n_h

    NB = 3
    mesh = plsc.VectorSubcoreMesh(core_axis_name="c", subcore_axis_name="s")

    @functools.partial(
        pl.kernel,
        out_type=jax.ShapeDtypeStruct((batch * seq, d_model), jnp.float32),
        mesh=mesh,
        scratch_types=[
            pltpu.VMEM((P, d_model), jnp.float32),
            [pltpu.VMEM((C, d_model), jnp.float32)] * NB,
            pltpu.VMEM((batch * P,), jnp.int32),
            [pltpu.SemaphoreType.DMA] * NB,
            [pltpu.SemaphoreType.DMA] * NB,
        ],
    )
    def body(x_hbm, table_hbm, pe_hbm, out_hbm,
             pe_v, toks, idx_all, gsems, ssems):
        wid = lax.axis_index("s") * _NC + lax.axis_index("c")
        pos0 = wid * P
        pltpu.sync_copy(pe_hbm.at[pl.ds(pos0, P)], pe_v)
        for b in range(batch):
            pltpu.sync_copy(x_hbm.at[pl.ds(b * seq + pos0, P)],
                            idx_all.at[pl.ds(b * P, P)])

        def offs(c):
            b, h = divmod(c, n_h)
            return b * P + h * C, b * seq + pos0 + h * C, h

        gathers, stores = {}, {}

        def issue_gather(c):
            pb = c % NB
            ioff, _, _ = offs(c)
            gathers[c] = pltpu.async_copy(
                table_hbm.at[idx_all.at[pl.ds(ioff, C)]], toks[pb], gsems[pb])

        issue_gather(0)
        if nchunks > 1:
            issue_gather(1)

        for c in range(nchunks):
            pb = c % NB
            if c + 2 < nchunks:
                if c >= 1:
                    stores[c - 1].wait()
                issue_gather(c + 2)
            gathers[c].wait()
            _, obase, h = offs(c)

            @pl.loop(0, C)
            def _(r):
                for col in range(0, d_model, _L):
                    plsc.addupdate(toks[pb].at[r, pl.ds(col, _L)],
                                   pe_v[h * C + r, pl.ds(col, _L)])

            stores[c] = pltpu.async_copy(
                toks[pb], out_hbm.at[pl.ds(obase, C)], ssems[pb])

        for c in range(max(0, nchunks - 3), nchunks):
            stores[c].wait()

    return body(x_flat, table, pe)


def kernel(x, token_table):
    batch, seq = x.shape
    d_model = token_table.shape[1]
    pe = jnp.asarray(_sinusoidal_pe_np(seq, d_model))
    x_flat = x.reshape(batch * seq).astype(jnp.int32)
    out = _embed(x_flat, token_table, pe,
                 batch=batch, seq=seq, d_model=d_model)
    return out.reshape(batch, seq, d_model)
